# baseline (speedup 1.0000x reference)
import functools
import math

import jax
import jax.numpy as jnp
from jax import lax
from jax.experimental import pallas as pl
from jax.experimental.pallas import tpu as pltpu
from jax.experimental.pallas import tpu_sc as plsc

D_MODEL = 64
SCALE = float(math.sqrt(D_MODEL))

NW = 32
CHUNK = 128
NBUF = 4


def _emb_kernel(n_chunks, W_hbm, idx_hbm, out_hbm, idx_v, gbuf, obuf, *sems):
    gsems = sems[:NBUF]
    ssems = sems[NBUF:]
    wid = lax.axis_index("s") * 2 + lax.axis_index("c")
    base = wid * (n_chunks * CHUNK)

    pltpu.sync_copy(idx_hbm.at[wid], idx_v)

    def start_gather(b, g):
        pltpu.make_async_copy(
            W_hbm.at[idx_v.at[g]], gbuf.at[b], gsems[b]).start()

    def wait_gather(b):
        pltpu.make_async_copy(
            W_hbm.at[idx_v.at[0]], gbuf.at[b], gsems[b]).wait()

    def start_scatter(b, g):
        pltpu.make_async_copy(
            obuf.at[b], out_hbm.at[pl.ds(base + g * CHUNK, CHUNK)],
            ssems[b]).start()

    def wait_scatter(b, g):
        pltpu.make_async_copy(
            obuf.at[b], out_hbm.at[pl.ds(base + g * CHUNK, CHUNK)],
            ssems[b]).wait()

    def scale(b):
        def row(r, carry):
            for c in range(D_MODEL // 16):
                obuf[b, r, pl.ds(c * 16, 16)] = (
                    gbuf[b, r, pl.ds(c * 16, 16)] * SCALE)
            return carry
        lax.fori_loop(0, CHUNK, row, 0, unroll=4)

    for b in range(NBUF):
        start_gather(b, g=b)

    for b in range(NBUF):
        wait_gather(b)
        scale(b)
        start_scatter(b, g=b)
        start_gather(b, g=b + NBUF)

    def round_body(g0, carry):
        for b in range(NBUF):
            g = g0 + b
            wait_gather(b)
            wait_scatter(b, g - NBUF)
            scale(b)
            start_scatter(b, g)
            start_gather(b, g + NBUF)
        return carry
    lax.fori_loop(1, n_chunks // NBUF - 1,
                  lambda i, c: round_body(i * NBUF, c), 0)

    for b in range(NBUF):
        g = n_chunks - NBUF + b
        wait_gather(b)
        wait_scatter(b, g - NBUF)
        scale(b)
        start_scatter(b, g)

    for b in range(NBUF):
        wait_scatter(b, n_chunks - NBUF + b)


def _build(n_chunks):
    mesh = plsc.VectorSubcoreMesh(core_axis_name="c", subcore_axis_name="s")
    B = NW * n_chunks * CHUNK
    return functools.partial(
        pl.kernel,
        mesh=mesh,
        out_type=jax.ShapeDtypeStruct((B, D_MODEL), jnp.float32),
        scratch_types=[
            pltpu.VMEM((n_chunks, CHUNK), jnp.int32),
            pltpu.VMEM((NBUF, CHUNK, D_MODEL), jnp.float32),
            pltpu.VMEM((NBUF, CHUNK, D_MODEL), jnp.float32),
        ] + [pltpu.SemaphoreType.DMA] * (2 * NBUF),
        compiler_params=pltpu.CompilerParams(use_tc_tiling_on_sc=False),
    )(functools.partial(_emb_kernel, n_chunks))


@jax.jit
def kernel(x, W):
    orig_shape = x.shape
    flat = x.reshape(-1).astype(jnp.int32)
    B = flat.shape[0]
    n_chunks = B // (NW * CHUNK)
    assert B == NW * n_chunks * CHUNK
    idx = flat.reshape(NW, n_chunks, CHUNK)
    out = _build(n_chunks)(W, idx)
    return out.reshape(*orig_shape, D_MODEL)


# speedup vs baseline: 1.0032x; 1.0032x over previous
import functools
import math

import jax
import jax.numpy as jnp
from jax import lax
from jax.experimental import pallas as pl
from jax.experimental.pallas import tpu as pltpu
from jax.experimental.pallas import tpu_sc as plsc

D_MODEL = 64
SCALE = float(math.sqrt(D_MODEL))

NW = 32
CHUNK = 128
NBUF = 4


def _emb_kernel(n_chunks, W_hbm, idx_hbm, out_hbm, idx_v, gbuf, obuf, *sems):
    gsems = sems[:NBUF]
    ssems = sems[NBUF:]
    wid = lax.axis_index("s") * 2 + lax.axis_index("c")
    base = wid * (n_chunks * CHUNK)

    pltpu.sync_copy(idx_hbm.at[wid], idx_v)

    def start_gather(b, g):
        pltpu.make_async_copy(
            W_hbm.at[idx_v.at[g]], gbuf.at[b], gsems[b]).start()

    def wait_gather(b):
        pltpu.make_async_copy(
            W_hbm.at[idx_v.at[0]], gbuf.at[b], gsems[b]).wait()

    def start_scatter(b, g):
        pltpu.make_async_copy(
            obuf.at[b], out_hbm.at[pl.ds(base + g * CHUNK, CHUNK)],
            ssems[b]).start()

    def wait_scatter(b, g):
        pltpu.make_async_copy(
            obuf.at[b], out_hbm.at[pl.ds(base + g * CHUNK, CHUNK)],
            ssems[b]).wait()

    def scale(b):
        def row(r, carry):
            for c in range(D_MODEL // 16):
                obuf[b, r, pl.ds(c * 16, 16)] = (
                    gbuf[b, r, pl.ds(c * 16, 16)] * SCALE)
            return carry
        lax.fori_loop(0, CHUNK, row, 0, unroll=4)

    for b in range(NBUF):
        start_gather(b, g=b)

    for b in range(NBUF):
        wait_gather(b)
        scale(b)
        start_scatter(b, g=b)
        start_gather(b, g=b + NBUF)

    def round_body(g0, carry):
        for b in range(NBUF):
            g = g0 + b
            wait_gather(b)
            wait_scatter(b, g - NBUF)
            scale(b)
            start_scatter(b, g)
            start_gather(b, g + NBUF)
        return carry
    lax.fori_loop(1, n_chunks // NBUF - 1,
                  lambda i, c: round_body(i * NBUF, c), 0)

    for b in range(NBUF):
        g = n_chunks - NBUF + b
        wait_gather(b)
        wait_scatter(b, g - NBUF)
        scale(b)
        start_scatter(b, g)

    for b in range(NBUF):
        wait_scatter(b, n_chunks - NBUF + b)


def _build(n_chunks):
    mesh = plsc.VectorSubcoreMesh(core_axis_name="c", subcore_axis_name="s")
    B = NW * n_chunks * CHUNK
    return functools.partial(
        pl.kernel,
        mesh=mesh,
        out_type=jax.ShapeDtypeStruct((B, D_MODEL), jnp.float32),
        scratch_types=[
            pltpu.VMEM((n_chunks, CHUNK), jnp.int32),
            pltpu.VMEM((NBUF, CHUNK, D_MODEL), jnp.float32),
            pltpu.VMEM((NBUF, CHUNK, D_MODEL), jnp.float32),
        ] + [pltpu.SemaphoreType.DMA] * (2 * NBUF),
        compiler_params=pltpu.CompilerParams(
            use_tc_tiling_on_sc=False, skip_device_barrier=True),
    )(functools.partial(_emb_kernel, n_chunks))


@jax.jit
def kernel(x, W):
    orig_shape = x.shape
    flat = x.reshape(-1).astype(jnp.int32)
    B = flat.shape[0]
    n_chunks = B // (NW * CHUNK)
    assert B == NW * n_chunks * CHUNK
    idx = flat.reshape(NW, n_chunks, CHUNK)
    out = _build(n_chunks)(W, idx)
    return out.reshape(*orig_shape, D_MODEL)
